# Initial kernel scaffold; baseline (speedup 1.0000x reference)
import jax
import jax.numpy as jnp
from jax.experimental import pallas as pl


def kernel(x, edge_index, edge_attr, batch, W1, b1, W2, b2):
    raise NotImplementedError("write your pallas kernel here")


# speedup vs baseline: 8.7403x; 8.7403x over previous
import functools

import jax
import jax.numpy as jnp
from jax import lax
from jax.experimental import pallas as pl
from jax.experimental.pallas import tpu as pltpu
from jax.experimental.pallas import tpu_sc as plsc

N = 50000
E = 800000
IN_CH = 6
HID = 64
HALF = HID // 2
G = 64
NB = 2000
GRID = N // NB

NCORE = 2
NSUB = 16
ROWS_PER_TILE = 3128
LAST_TILE_BASE = N - ROWS_PER_TILE
E_PER_TILE = E // NSUB
MP_CHUNK = 80
MP_ITERS = E_PER_TILE // MP_CHUNK
E_PER_CORE = E // NCORE
E_PER_DTILE = E_PER_CORE // NSUB
DG_CHUNK = 40
DG_ITERS = E_PER_DTILE // DG_CHUNK

_F32 = jnp.float32
_PREC = jax.lax.Precision.HIGHEST


def _sc_mesh():
    return plsc.VectorSubcoreMesh(core_axis_name="c", subcore_axis_name="s")


def _deg_body(dst_hbm, zrow_hbm, deg_out, idx_v, ones_v, stage_v, acc, _):
    c = lax.axis_index("c")
    s = lax.axis_index("s")
    for k in range(3):
        ones_v[pl.ds(k * 16, 16)] = jnp.ones((16,), _F32)
    nbase = jnp.where(s == NSUB - 1, LAST_TILE_BASE, s * ROWS_PER_TILE)
    pltpu.sync_copy(zrow_hbm, stage_v)
    pltpu.sync_copy(stage_v, acc.at[pl.ds(nbase, ROWS_PER_TILE)])
    plsc.subcore_barrier()
    ebase = c * E_PER_CORE + s * E_PER_DTILE

    def step(j, carry):
        eb = ebase + j * DG_CHUNK
        pltpu.sync_copy(dst_hbm.at[pl.ds(eb, DG_CHUNK)], idx_v)
        pltpu.sync_copy(ones_v.at[pl.ds(0, DG_CHUNK)], acc.at[idx_v], add=True)
        return carry

    lax.fori_loop(0, DG_ITERS, step, 0)
    plsc.subcore_barrier()
    pltpu.sync_copy(acc.at[pl.ds(nbase, ROWS_PER_TILE)], stage_v)
    pltpu.sync_copy(stage_v, deg_out.at[pl.ds(c * N + nbase, ROWS_PER_TILE)])


def _deg_call(dst, zrow1d):
    return pl.kernel(
        _deg_body,
        out_type=jax.ShapeDtypeStruct((NCORE * N,), _F32),
        mesh=_sc_mesh(),
        scratch_types=[
            pltpu.VMEM((DG_CHUNK,), jnp.int32),
            pltpu.VMEM((48,), _F32),
            pltpu.VMEM((ROWS_PER_TILE,), _F32),
            pltpu.VMEM_SHARED((N,), _F32),
            pltpu.SemaphoreType.DMA,
        ],
    )(dst, zrow1d)


STAGE_ROWS = 184
STAGE_ITERS = ROWS_PER_TILE // STAGE_ROWS


def _mp_body(y0_hbm, y1_hbm, src_hbm, dst_hbm, zrows_hbm, s_out,
             idx_s, idx_d, rows_v, stage_v, acc, sem):
    c = lax.axis_index("c")
    s = lax.axis_index("s")
    nbase = jnp.where(s == NSUB - 1, LAST_TILE_BASE, s * ROWS_PER_TILE)
    pltpu.sync_copy(zrows_hbm, stage_v)

    def zinit(k, carry):
        pltpu.sync_copy(stage_v, acc.at[pl.ds(nbase + k * STAGE_ROWS,
                                              STAGE_ROWS)])
        return carry

    lax.fori_loop(0, STAGE_ITERS, zinit, 0)
    plsc.subcore_barrier()
    ebase = s * E_PER_TILE

    def run(y_hbm):
        def step(j, carry):
            eb = ebase + j * MP_CHUNK
            pltpu.sync_copy(src_hbm.at[pl.ds(eb, MP_CHUNK)], idx_s)
            pltpu.async_copy(y_hbm.at[idx_s], rows_v, sem).wait()
            pltpu.sync_copy(dst_hbm.at[pl.ds(eb, MP_CHUNK)], idx_d)
            pltpu.sync_copy(rows_v, acc.at[idx_d], add=True)
            return carry
        lax.fori_loop(0, MP_ITERS, step, 0)

    pl.when(c == 0)(lambda: run(y0_hbm))
    pl.when(c == 1)(lambda: run(y1_hbm))
    plsc.subcore_barrier()

    def copyout(k, carry):
        rb = nbase + k * STAGE_ROWS
        pltpu.sync_copy(acc.at[pl.ds(rb, STAGE_ROWS)], stage_v)
        pltpu.sync_copy(stage_v, s_out.at[c, pl.ds(rb, STAGE_ROWS)])
        return carry

    lax.fori_loop(0, STAGE_ITERS, copyout, 0)


def _mp_call(y0, y1, src, dst, zrows):
    return pl.kernel(
        _mp_body,
        out_type=jax.ShapeDtypeStruct((NCORE, N, HALF), _F32),
        mesh=_sc_mesh(),
        scratch_types=[
            pltpu.VMEM((MP_CHUNK,), jnp.int32),
            pltpu.VMEM((MP_CHUNK,), jnp.int32),
            pltpu.VMEM((MP_CHUNK, HALF), _F32),
            pltpu.VMEM((STAGE_ROWS, HALF), _F32),
            pltpu.VMEM_SHARED((N, HALF), _F32),
            pltpu.SemaphoreType.DMA,
        ],
        compiler_params=pltpu.CompilerParams(use_tc_tiling_on_sc=False),
    )(y0, y1, src, dst, zrows)


def _prep1_body(degp_ref, x_ref, w1_ref, y0_ref, y1_ref, dinv_ref):
    deg = degp_ref[0] + degp_ref[1] + 1.0
    dinv = lax.rsqrt(deg)
    xw = jnp.dot(x_ref[...], w1_ref[...], preferred_element_type=_F32,
                 precision=_PREC)
    y = xw * dinv
    y0_ref[...] = y[:, :HALF]
    y1_ref[...] = y[:, HALF:]
    dinv_ref[...] = dinv


def _prep1_call(degp, x, w1):
    return pl.pallas_call(
        _prep1_body,
        grid=(GRID,),
        in_specs=[
            pl.BlockSpec((NCORE, NB, 1), lambda i: (0, i, 0)),
            pl.BlockSpec((NB, IN_CH), lambda i: (i, 0)),
            pl.BlockSpec((IN_CH, HID), lambda i: (0, 0)),
        ],
        out_specs=[
            pl.BlockSpec((NB, HALF), lambda i: (i, 0)),
            pl.BlockSpec((NB, HALF), lambda i: (i, 0)),
            pl.BlockSpec((NB, 1), lambda i: (i, 0)),
        ],
        out_shape=[
            jax.ShapeDtypeStruct((N, HALF), _F32),
            jax.ShapeDtypeStruct((N, HALF), _F32),
            jax.ShapeDtypeStruct((N, 1), _F32),
        ],
    )(degp, x, w1)


def _prep2_body(s1_ref, y0_ref, y1_ref, dinv_ref, w2_ref, b1_ref,
                y20_ref, y21_ref):
    dinv = dinv_ref[...]
    t = jnp.concatenate([s1_ref[0] + y0_ref[...], s1_ref[1] + y1_ref[...]],
                        axis=1)
    h = jnp.maximum(t * dinv + b1_ref[...], 0.0)
    y2 = jnp.dot(h, w2_ref[...], preferred_element_type=_F32,
                 precision=_PREC) * dinv
    y20_ref[...] = y2[:, :HALF]
    y21_ref[...] = y2[:, HALF:]


def _prep2_call(s1, y0, y1, dinv, w2, b1):
    return pl.pallas_call(
        _prep2_body,
        grid=(GRID,),
        in_specs=[
            pl.BlockSpec((NCORE, NB, HALF), lambda i: (0, i, 0)),
            pl.BlockSpec((NB, HALF), lambda i: (i, 0)),
            pl.BlockSpec((NB, HALF), lambda i: (i, 0)),
            pl.BlockSpec((NB, 1), lambda i: (i, 0)),
            pl.BlockSpec((HID, HID), lambda i: (0, 0)),
            pl.BlockSpec((1, HID), lambda i: (0, 0)),
        ],
        out_specs=[
            pl.BlockSpec((NB, HALF), lambda i: (i, 0)),
            pl.BlockSpec((NB, HALF), lambda i: (i, 0)),
        ],
        out_shape=[
            jax.ShapeDtypeStruct((N, HALF), _F32),
            jax.ShapeDtypeStruct((N, HALF), _F32),
        ],
    )(s1, y0, y1, dinv, w2, b1)


def _pool_body(s2_ref, y20_ref, y21_ref, dinv_ref, b2_ref, batch_ref,
               out_ref, acc_ref):
    i = pl.program_id(0)
    t = jnp.concatenate([s2_ref[0] + y20_ref[...], s2_ref[1] + y21_ref[...]],
                        axis=1)
    h2 = t * dinv_ref[...] + b2_ref[...]
    h2e = jnp.concatenate([h2, jnp.ones((NB, HID), _F32)], axis=1)
    oht = (lax.broadcasted_iota(jnp.int32, (G, NB), 0) ==
           batch_ref[0]).astype(_F32)
    p = jnp.dot(oht, h2e, preferred_element_type=_F32, precision=_PREC)

    @pl.when(i == 0)
    def _():
        acc_ref[...] = p

    @pl.when(i > 0)
    def _():
        acc_ref[...] += p

    @pl.when(i == GRID - 1)
    def _():
        a = acc_ref[...]
        counts = a[:, HID:HID + 1]
        out_ref[...] = a[:, :HID] / jnp.maximum(counts, 1.0)


def _pool_call(s2, y20, y21, dinv, b2, batch2d):
    return pl.pallas_call(
        _pool_body,
        grid=(GRID,),
        in_specs=[
            pl.BlockSpec((NCORE, NB, HALF), lambda i: (0, i, 0)),
            pl.BlockSpec((NB, HALF), lambda i: (i, 0)),
            pl.BlockSpec((NB, HALF), lambda i: (i, 0)),
            pl.BlockSpec((NB, 1), lambda i: (i, 0)),
            pl.BlockSpec((1, HID), lambda i: (0, 0)),
            pl.BlockSpec((1, 1, NB), lambda i: (i, 0, 0)),
        ],
        out_specs=pl.BlockSpec((G, HID), lambda i: (0, 0)),
        out_shape=jax.ShapeDtypeStruct((G, HID), _F32),
        scratch_shapes=[pltpu.VMEM((G, 2 * HID), _F32)],
    )(s2, y20, y21, dinv, b2, batch2d)


@functools.partial(jax.jit)
def kernel(x, edge_index, edge_attr, batch, W1, b1, W2, b2):
    del edge_attr
    src = edge_index[0].astype(jnp.int32)
    dst = edge_index[1].astype(jnp.int32)
    batch2d = batch.astype(jnp.int32).reshape(GRID, 1, NB)
    zrow1d = jnp.zeros((ROWS_PER_TILE,), _F32)
    zrows = jnp.zeros((STAGE_ROWS, HALF), _F32)
    b1r = b1.reshape(1, HID)
    b2r = b2.reshape(1, HID)

    degp = _deg_call(dst, zrow1d)
    y0, y1, dinv = _prep1_call(degp.reshape(NCORE, N, 1), x, W1)
    s1 = _mp_call(y0, y1, src, dst, zrows)
    y20, y21 = _prep2_call(s1, y0, y1, dinv, W2, b1r)
    s2 = _mp_call(y20, y21, src, dst, zrows)
    return _pool_call(s2, y20, y21, dinv, b2r, batch2d)


# speedup vs baseline: 19.2677x; 2.2045x over previous
import functools

import jax
import jax.numpy as jnp
from jax import lax
from jax.experimental import pallas as pl
from jax.experimental.pallas import tpu as pltpu
from jax.experimental.pallas import tpu_sc as plsc

N = 50000
E = 800000
IN_CH = 6
HID = 64
HALF = HID // 2
G = 64
NB = 2000
GRID = N // NB

NCORE = 2
NSUB = 16
ROWS_PER_TILE = 3128
LAST_TILE_BASE = N - ROWS_PER_TILE
E_PER_TILE = E // NSUB
MP_CHUNK = 80
MP_ITERS = E_PER_TILE // MP_CHUNK
E_PER_CORE = E // NCORE
E_PER_DTILE = E_PER_CORE // NSUB
DG_CHUNK = 40
DG_ITERS = E_PER_DTILE // DG_CHUNK

_F32 = jnp.float32
_PREC = jax.lax.Precision.HIGHEST


def _sc_mesh():
    return plsc.VectorSubcoreMesh(core_axis_name="c", subcore_axis_name="s")


def _deg_body(dst_hbm, zrow_hbm, deg_out, idx_v, ones_v, stage_v, acc, _):
    c = lax.axis_index("c")
    s = lax.axis_index("s")
    for k in range(3):
        ones_v[pl.ds(k * 16, 16)] = jnp.ones((16,), _F32)
    nbase = jnp.where(s == NSUB - 1, LAST_TILE_BASE, s * ROWS_PER_TILE)
    pltpu.sync_copy(zrow_hbm, stage_v)
    pltpu.sync_copy(stage_v, acc.at[pl.ds(nbase, ROWS_PER_TILE)])
    plsc.subcore_barrier()
    ebase = c * E_PER_CORE + s * E_PER_DTILE

    def step(j, carry):
        eb = ebase + j * DG_CHUNK
        pltpu.sync_copy(dst_hbm.at[pl.ds(eb, DG_CHUNK)], idx_v)
        pltpu.sync_copy(ones_v.at[pl.ds(0, DG_CHUNK)], acc.at[idx_v], add=True)
        return carry

    lax.fori_loop(0, DG_ITERS, step, 0)
    plsc.subcore_barrier()
    pltpu.sync_copy(acc.at[pl.ds(nbase, ROWS_PER_TILE)], stage_v)
    pltpu.sync_copy(stage_v, deg_out.at[pl.ds(c * N + nbase, ROWS_PER_TILE)])


def _deg_call(dst, zrow1d):
    return pl.kernel(
        _deg_body,
        out_type=jax.ShapeDtypeStruct((NCORE * N,), _F32),
        mesh=_sc_mesh(),
        scratch_types=[
            pltpu.VMEM((DG_CHUNK,), jnp.int32),
            pltpu.VMEM((48,), _F32),
            pltpu.VMEM((ROWS_PER_TILE,), _F32),
            pltpu.VMEM_SHARED((N,), _F32),
            pltpu.SemaphoreType.DMA,
        ],
    )(dst, zrow1d)


STAGE_ROWS = 136
STAGE_ITERS = ROWS_PER_TILE // STAGE_ROWS
SUP_CHUNKS = 5
SUP_EDGES = SUP_CHUNKS * MP_CHUNK
N_SUPER = E_PER_TILE // SUP_EDGES


def _mp_body(y0_hbm, y1_hbm, src_hbm, dst2d_hbm, zrows_hbm, s_out,
             srcv, dstv, rows_v, acc, sem):
    c = lax.axis_index("c")
    s = lax.axis_index("s")
    nbase = jnp.where(s == NSUB - 1, LAST_TILE_BASE, s * ROWS_PER_TILE)
    stage_v = rows_v.at[0, pl.ds(0, STAGE_ROWS)]
    pltpu.sync_copy(zrows_hbm, stage_v)

    def zinit(k, carry):
        pltpu.sync_copy(stage_v, acc.at[pl.ds(nbase + k * STAGE_ROWS,
                                              STAGE_ROWS)])
        return carry

    lax.fori_loop(0, STAGE_ITERS, zinit, 0)
    plsc.subcore_barrier()
    ebase = s * E_PER_TILE
    cbase = s * (E_PER_TILE // MP_CHUNK)

    def run(y_hbm):
        def load_and_fire(i, b):
            pltpu.sync_copy(src_hbm.at[pl.ds(ebase + i * SUP_EDGES,
                                             SUP_EDGES)],
                            srcv.at[b])
            pltpu.sync_copy(dst2d_hbm.at[pl.ds(cbase + i * SUP_CHUNKS,
                                               SUP_CHUNKS)],
                            dstv.at[b])
            for k in range(SUP_CHUNKS):
                pltpu.async_copy(
                    y_hbm.at[srcv.at[b, pl.ds(k * MP_CHUNK, MP_CHUNK)]],
                    rows_v.at[b, pl.ds(k * MP_CHUNK, MP_CHUNK)], sem)

        load_and_fire(0, 0)

        def step(i, carry):
            b = lax.rem(i, 2)
            for k in range(SUP_CHUNKS):
                pltpu.make_async_copy(
                    y_hbm.at[srcv.at[b, pl.ds(k * MP_CHUNK, MP_CHUNK)]],
                    rows_v.at[b, pl.ds(k * MP_CHUNK, MP_CHUNK)], sem).wait()
            pl.when(i + 1 < N_SUPER)(lambda: load_and_fire(i + 1, 1 - b))
            for k in range(SUP_CHUNKS):
                pltpu.sync_copy(
                    rows_v.at[b, pl.ds(k * MP_CHUNK, MP_CHUNK)],
                    acc.at[dstv.at[b, k]], add=True)
            return carry

        lax.fori_loop(0, N_SUPER, step, 0)

    pl.when(c == 0)(lambda: run(y0_hbm))
    pl.when(c == 1)(lambda: run(y1_hbm))
    plsc.subcore_barrier()

    def copyout(k, carry):
        rb = nbase + k * STAGE_ROWS
        pltpu.sync_copy(acc.at[pl.ds(rb, STAGE_ROWS)], stage_v)
        pltpu.sync_copy(stage_v, s_out.at[c, pl.ds(rb, STAGE_ROWS)])
        return carry

    lax.fori_loop(0, STAGE_ITERS, copyout, 0)


def _mp_call(y0, y1, src, dst2d, zrows):
    return pl.kernel(
        _mp_body,
        out_type=jax.ShapeDtypeStruct((NCORE, N, HALF), _F32),
        mesh=_sc_mesh(),
        scratch_types=[
            pltpu.VMEM((2, SUP_EDGES), jnp.int32),
            pltpu.VMEM((2, SUP_CHUNKS, MP_CHUNK), jnp.int32),
            pltpu.VMEM((2, SUP_EDGES, HALF), _F32),
            pltpu.VMEM_SHARED((N, HALF), _F32),
            pltpu.SemaphoreType.DMA,
        ],
        compiler_params=pltpu.CompilerParams(use_tc_tiling_on_sc=False),
    )(y0, y1, src, dst2d, zrows)


def _prep1_body(degp_ref, x_ref, w1_ref, y0_ref, y1_ref, dinv_ref):
    deg = degp_ref[0] + degp_ref[1] + 1.0
    dinv = lax.rsqrt(deg)
    xw = jnp.dot(x_ref[...], w1_ref[...], preferred_element_type=_F32,
                 precision=_PREC)
    y = xw * dinv
    y0_ref[...] = y[:, :HALF]
    y1_ref[...] = y[:, HALF:]
    dinv_ref[...] = dinv


def _prep1_call(degp, x, w1):
    return pl.pallas_call(
        _prep1_body,
        grid=(GRID,),
        in_specs=[
            pl.BlockSpec((NCORE, NB, 1), lambda i: (0, i, 0)),
            pl.BlockSpec((NB, IN_CH), lambda i: (i, 0)),
            pl.BlockSpec((IN_CH, HID), lambda i: (0, 0)),
        ],
        out_specs=[
            pl.BlockSpec((NB, HALF), lambda i: (i, 0)),
            pl.BlockSpec((NB, HALF), lambda i: (i, 0)),
            pl.BlockSpec((NB, 1), lambda i: (i, 0)),
        ],
        out_shape=[
            jax.ShapeDtypeStruct((N, HALF), _F32),
            jax.ShapeDtypeStruct((N, HALF), _F32),
            jax.ShapeDtypeStruct((N, 1), _F32),
        ],
    )(degp, x, w1)


def _prep2_body(s1_ref, y0_ref, y1_ref, dinv_ref, w2_ref, b1_ref,
                y20_ref, y21_ref):
    dinv = dinv_ref[...]
    t = jnp.concatenate([s1_ref[0] + y0_ref[...], s1_ref[1] + y1_ref[...]],
                        axis=1)
    h = jnp.maximum(t * dinv + b1_ref[...], 0.0)
    y2 = jnp.dot(h, w2_ref[...], preferred_element_type=_F32,
                 precision=_PREC) * dinv
    y20_ref[...] = y2[:, :HALF]
    y21_ref[...] = y2[:, HALF:]


def _prep2_call(s1, y0, y1, dinv, w2, b1):
    return pl.pallas_call(
        _prep2_body,
        grid=(GRID,),
        in_specs=[
            pl.BlockSpec((NCORE, NB, HALF), lambda i: (0, i, 0)),
            pl.BlockSpec((NB, HALF), lambda i: (i, 0)),
            pl.BlockSpec((NB, HALF), lambda i: (i, 0)),
            pl.BlockSpec((NB, 1), lambda i: (i, 0)),
            pl.BlockSpec((HID, HID), lambda i: (0, 0)),
            pl.BlockSpec((1, HID), lambda i: (0, 0)),
        ],
        out_specs=[
            pl.BlockSpec((NB, HALF), lambda i: (i, 0)),
            pl.BlockSpec((NB, HALF), lambda i: (i, 0)),
        ],
        out_shape=[
            jax.ShapeDtypeStruct((N, HALF), _F32),
            jax.ShapeDtypeStruct((N, HALF), _F32),
        ],
    )(s1, y0, y1, dinv, w2, b1)


def _pool_body(s2_ref, y20_ref, y21_ref, dinv_ref, b2_ref, batch_ref,
               out_ref, acc_ref):
    i = pl.program_id(0)
    t = jnp.concatenate([s2_ref[0] + y20_ref[...], s2_ref[1] + y21_ref[...]],
                        axis=1)
    h2 = t * dinv_ref[...] + b2_ref[...]
    h2e = jnp.concatenate([h2, jnp.ones((NB, HID), _F32)], axis=1)
    oht = (lax.broadcasted_iota(jnp.int32, (G, NB), 0) ==
           batch_ref[0]).astype(_F32)
    p = jnp.dot(oht, h2e, preferred_element_type=_F32, precision=_PREC)

    @pl.when(i == 0)
    def _():
        acc_ref[...] = p

    @pl.when(i > 0)
    def _():
        acc_ref[...] += p

    @pl.when(i == GRID - 1)
    def _():
        a = acc_ref[...]
        counts = a[:, HID:HID + 1]
        out_ref[...] = a[:, :HID] / jnp.maximum(counts, 1.0)


def _pool_call(s2, y20, y21, dinv, b2, batch2d):
    return pl.pallas_call(
        _pool_body,
        grid=(GRID,),
        in_specs=[
            pl.BlockSpec((NCORE, NB, HALF), lambda i: (0, i, 0)),
            pl.BlockSpec((NB, HALF), lambda i: (i, 0)),
            pl.BlockSpec((NB, HALF), lambda i: (i, 0)),
            pl.BlockSpec((NB, 1), lambda i: (i, 0)),
            pl.BlockSpec((1, HID), lambda i: (0, 0)),
            pl.BlockSpec((1, 1, NB), lambda i: (i, 0, 0)),
        ],
        out_specs=pl.BlockSpec((G, HID), lambda i: (0, 0)),
        out_shape=jax.ShapeDtypeStruct((G, HID), _F32),
        scratch_shapes=[pltpu.VMEM((G, 2 * HID), _F32)],
    )(s2, y20, y21, dinv, b2, batch2d)


@functools.partial(jax.jit)
def kernel(x, edge_index, edge_attr, batch, W1, b1, W2, b2):
    del edge_attr
    src = edge_index[0].astype(jnp.int32)
    dst = edge_index[1].astype(jnp.int32)
    batch2d = batch.astype(jnp.int32).reshape(GRID, 1, NB)
    zrow1d = jnp.zeros((ROWS_PER_TILE,), _F32)
    zrows = jnp.zeros((STAGE_ROWS, HALF), _F32)
    b1r = b1.reshape(1, HID)
    b2r = b2.reshape(1, HID)

    dst2d = dst.reshape(E // MP_CHUNK, MP_CHUNK)
    degp = _deg_call(dst, zrow1d)
    y0, y1, dinv = _prep1_call(degp.reshape(NCORE, N, 1), x, W1)
    s1 = _mp_call(y0, y1, src, dst2d, zrows)
    y20, y21 = _prep2_call(s1, y0, y1, dinv, W2, b1r)
    s2 = _mp_call(y20, y21, src, dst2d, zrows)
    return _pool_call(s2, y20, y21, dinv, b2r, batch2d)


# speedup vs baseline: 31.1124x; 1.6147x over previous
import functools

import jax
import jax.numpy as jnp
from jax import lax
from jax.experimental import pallas as pl
from jax.experimental.pallas import tpu as pltpu
from jax.experimental.pallas import tpu_sc as plsc

N = 50000
E = 800000
IN_CH = 6
HID = 64
HALF = HID // 2
G = 64
NB = 2000
GRID = N // NB

NCORE = 2
NSUB = 16
ROWS_PER_TILE = 3128
LAST_TILE_BASE = N - ROWS_PER_TILE
E_PER_TILE = E // NSUB
MP_CHUNK = 80
MP_ITERS = E_PER_TILE // MP_CHUNK
E_PER_CORE = E // NCORE
E_PER_DTILE = E_PER_CORE // NSUB
DG_CHUNK = 40
DG_ITERS = E_PER_DTILE // DG_CHUNK

_F32 = jnp.float32
_PREC = jax.lax.Precision.HIGHEST


def _sc_mesh():
    return plsc.VectorSubcoreMesh(core_axis_name="c", subcore_axis_name="s")


DG_SUP = 5
DG_SUP_EDGES = DG_SUP * DG_CHUNK
DG_NSUP = E_PER_DTILE // DG_SUP_EDGES


def _deg_body(dstd_hbm, zrow_hbm, deg_out, idxd, ones_v, stage_v, acc,
              sem_i, sem_s):
    c = lax.axis_index("c")
    s = lax.axis_index("s")
    for k in range(3):
        ones_v[pl.ds(k * 16, 16)] = jnp.ones((16,), _F32)
    nbase = jnp.where(s == NSUB - 1, LAST_TILE_BASE, s * ROWS_PER_TILE)
    pltpu.sync_copy(zrow_hbm, stage_v)
    pltpu.sync_copy(stage_v, acc.at[pl.ds(nbase, ROWS_PER_TILE)])
    plsc.subcore_barrier()
    rbase = (c * NSUB + s) * DG_NSUP

    def idx_copy(i):
        return pltpu.make_async_copy(dstd_hbm.at[rbase + i],
                                     idxd.at[lax.rem(i, 3)], sem_i)

    def _sc_refs(i, k):
        return (ones_v.at[pl.ds(0, DG_CHUNK)],
                acc.at[idxd.at[lax.rem(i, 3), k]])

    class sc_copy:
        def __init__(self, i, k):
            self.i, self.k = i, k

        def start(self):
            pltpu.async_copy(*_sc_refs(self.i, self.k), sem_s, add=True)

        def wait(self):
            pltpu.make_async_copy(*_sc_refs(self.i, self.k), sem_s).wait()

    idx_copy(0).start()
    idx_copy(1).start()

    def step(i, carry):
        idx_copy(i).wait()

        @pl.when(i > 0)
        def _():
            for k in range(DG_SUP):
                sc_copy(i - 1, k).wait()

        @pl.when(i + 2 < DG_NSUP)
        def _():
            idx_copy(i + 2).start()

        for k in range(DG_SUP):
            sc_copy(i, k).start()
        return carry

    lax.fori_loop(0, DG_NSUP, step, 0)
    for k in range(DG_SUP):
        sc_copy(DG_NSUP - 1, k).wait()
    plsc.subcore_barrier()
    pltpu.sync_copy(acc.at[pl.ds(nbase, ROWS_PER_TILE)], stage_v)
    pltpu.sync_copy(stage_v, deg_out.at[pl.ds(c * N + nbase, ROWS_PER_TILE)])


def _deg_call(dstd, zrow1d):
    return pl.kernel(
        _deg_body,
        out_type=jax.ShapeDtypeStruct((NCORE * N,), _F32),
        mesh=_sc_mesh(),
        scratch_types=[
            pltpu.VMEM((3, DG_SUP, DG_CHUNK), jnp.int32),
            pltpu.VMEM((48,), _F32),
            pltpu.VMEM((ROWS_PER_TILE,), _F32),
            pltpu.VMEM_SHARED((N,), _F32),
            pltpu.SemaphoreType.DMA,
            pltpu.SemaphoreType.DMA,
        ],
        compiler_params=pltpu.CompilerParams(use_tc_tiling_on_sc=False),
    )(dstd, zrow1d)


STAGE_ROWS = 136
STAGE_ITERS = ROWS_PER_TILE // STAGE_ROWS
SUP_CHUNKS = 5
SUP_EDGES = SUP_CHUNKS * MP_CHUNK
N_SUPER = E_PER_TILE // SUP_EDGES


def _mp_body(y0_hbm, y1_hbm, pidx_hbm, zrows_hbm, s_out,
             pidxv, rows_v, acc, sem_i, sem_g, sem_s):
    c = lax.axis_index("c")
    s = lax.axis_index("s")
    nbase = jnp.where(s == NSUB - 1, LAST_TILE_BASE, s * ROWS_PER_TILE)
    stage_v = rows_v.at[0, pl.ds(0, STAGE_ROWS)]
    pltpu.sync_copy(zrows_hbm, stage_v)

    def zinit(k, carry):
        pltpu.sync_copy(stage_v, acc.at[pl.ds(nbase + k * STAGE_ROWS,
                                              STAGE_ROWS)])
        return carry

    lax.fori_loop(0, STAGE_ITERS, zinit, 0)
    plsc.subcore_barrier()
    rbase = s * N_SUPER

    def run(y_hbm):
        def idx_copy(i):
            return pltpu.make_async_copy(pidx_hbm.at[rbase + i],
                                         pidxv.at[lax.rem(i, 3)], sem_i)

        def g_copy(i, k):
            return pltpu.make_async_copy(
                y_hbm.at[pidxv.at[lax.rem(i, 3), k]],
                rows_v.at[lax.rem(i, 2), pl.ds(k * MP_CHUNK, MP_CHUNK)],
                sem_g)

        def _s_refs(i, k):
            return (rows_v.at[lax.rem(i, 2), pl.ds(k * MP_CHUNK, MP_CHUNK)],
                    acc.at[pidxv.at[lax.rem(i, 3), SUP_CHUNKS + k]])

        class s_copy:
            def __init__(self, i, k):
                self.i, self.k = i, k

            def start(self):
                pltpu.async_copy(*_s_refs(self.i, self.k), sem_s, add=True)

            def wait(self):
                pltpu.make_async_copy(*_s_refs(self.i, self.k), sem_s).wait()

        idx_copy(0).start()
        idx_copy(1).start()
        idx_copy(0).wait()
        for k in range(SUP_CHUNKS):
            g_copy(0, k).start()

        def step(i, carry):
            @pl.when(i + 1 < N_SUPER)
            def _():
                idx_copy(i + 1).wait()

            @pl.when(i > 0)
            def _():
                for k in range(SUP_CHUNKS):
                    s_copy(i - 1, k).wait()

            @pl.when(i + 2 < N_SUPER)
            def _():
                idx_copy(i + 2).start()

            for k in range(SUP_CHUNKS):
                g_copy(i, k).wait()

            @pl.when(i + 1 < N_SUPER)
            def _():
                for k in range(SUP_CHUNKS):
                    g_copy(i + 1, k).start()

            for k in range(SUP_CHUNKS):
                s_copy(i, k).start()
            return carry

        lax.fori_loop(0, N_SUPER, step, 0)
        for k in range(SUP_CHUNKS):
            s_copy(N_SUPER - 1, k).wait()

    pl.when(c == 0)(lambda: run(y0_hbm))
    pl.when(c == 1)(lambda: run(y1_hbm))
    plsc.subcore_barrier()

    def copyout(k, carry):
        rb = nbase + k * STAGE_ROWS
        pltpu.sync_copy(acc.at[pl.ds(rb, STAGE_ROWS)], stage_v)
        pltpu.sync_copy(stage_v, s_out.at[c, pl.ds(rb, STAGE_ROWS)])
        return carry

    lax.fori_loop(0, STAGE_ITERS, copyout, 0)


def _mp_call(y0, y1, pidx, zrows):
    return pl.kernel(
        _mp_body,
        out_type=jax.ShapeDtypeStruct((NCORE, N, HALF), _F32),
        mesh=_sc_mesh(),
        scratch_types=[
            pltpu.VMEM((3, 2 * SUP_CHUNKS, MP_CHUNK), jnp.int32),
            pltpu.VMEM((2, SUP_EDGES, HALF), _F32),
            pltpu.VMEM_SHARED((N, HALF), _F32),
            pltpu.SemaphoreType.DMA,
            pltpu.SemaphoreType.DMA,
            pltpu.SemaphoreType.DMA,
        ],
        compiler_params=pltpu.CompilerParams(use_tc_tiling_on_sc=False),
    )(y0, y1, pidx, zrows)


def _prep1_body(degp_ref, x_ref, w1_ref, y0_ref, y1_ref, dinv_ref):
    deg = degp_ref[0] + degp_ref[1] + 1.0
    dinv = lax.rsqrt(deg)
    xw = jnp.dot(x_ref[...], w1_ref[...], preferred_element_type=_F32,
                 precision=_PREC)
    y = xw * dinv
    y0_ref[...] = y[:, :HALF]
    y1_ref[...] = y[:, HALF:]
    dinv_ref[...] = dinv


def _prep1_call(degp, x, w1):
    return pl.pallas_call(
        _prep1_body,
        grid=(GRID,),
        in_specs=[
            pl.BlockSpec((NCORE, NB, 1), lambda i: (0, i, 0)),
            pl.BlockSpec((NB, IN_CH), lambda i: (i, 0)),
            pl.BlockSpec((IN_CH, HID), lambda i: (0, 0)),
        ],
        out_specs=[
            pl.BlockSpec((NB, HALF), lambda i: (i, 0)),
            pl.BlockSpec((NB, HALF), lambda i: (i, 0)),
            pl.BlockSpec((NB, 1), lambda i: (i, 0)),
        ],
        out_shape=[
            jax.ShapeDtypeStruct((N, HALF), _F32),
            jax.ShapeDtypeStruct((N, HALF), _F32),
            jax.ShapeDtypeStruct((N, 1), _F32),
        ],
    )(degp, x, w1)


def _prep2_body(s1_ref, y0_ref, y1_ref, dinv_ref, w2_ref, b1_ref,
                y20_ref, y21_ref):
    dinv = dinv_ref[...]
    t = jnp.concatenate([s1_ref[0] + y0_ref[...], s1_ref[1] + y1_ref[...]],
                        axis=1)
    h = jnp.maximum(t * dinv + b1_ref[...], 0.0)
    y2 = jnp.dot(h, w2_ref[...], preferred_element_type=_F32,
                 precision=_PREC) * dinv
    y20_ref[...] = y2[:, :HALF]
    y21_ref[...] = y2[:, HALF:]


def _prep2_call(s1, y0, y1, dinv, w2, b1):
    return pl.pallas_call(
        _prep2_body,
        grid=(GRID,),
        in_specs=[
            pl.BlockSpec((NCORE, NB, HALF), lambda i: (0, i, 0)),
            pl.BlockSpec((NB, HALF), lambda i: (i, 0)),
            pl.BlockSpec((NB, HALF), lambda i: (i, 0)),
            pl.BlockSpec((NB, 1), lambda i: (i, 0)),
            pl.BlockSpec((HID, HID), lambda i: (0, 0)),
            pl.BlockSpec((1, HID), lambda i: (0, 0)),
        ],
        out_specs=[
            pl.BlockSpec((NB, HALF), lambda i: (i, 0)),
            pl.BlockSpec((NB, HALF), lambda i: (i, 0)),
        ],
        out_shape=[
            jax.ShapeDtypeStruct((N, HALF), _F32),
            jax.ShapeDtypeStruct((N, HALF), _F32),
        ],
    )(s1, y0, y1, dinv, w2, b1)


def _pool_body(s2_ref, y20_ref, y21_ref, dinv_ref, b2_ref, batch_ref,
               out_ref, acc_ref):
    i = pl.program_id(0)
    t = jnp.concatenate([s2_ref[0] + y20_ref[...], s2_ref[1] + y21_ref[...]],
                        axis=1)
    h2 = t * dinv_ref[...] + b2_ref[...]
    h2e = jnp.concatenate([h2, jnp.ones((NB, HID), _F32)], axis=1)
    oht = (lax.broadcasted_iota(jnp.int32, (G, NB), 0) ==
           batch_ref[0]).astype(_F32)
    p = jnp.dot(oht, h2e, preferred_element_type=_F32, precision=_PREC)

    @pl.when(i == 0)
    def _():
        acc_ref[...] = p

    @pl.when(i > 0)
    def _():
        acc_ref[...] += p

    @pl.when(i == GRID - 1)
    def _():
        a = acc_ref[...]
        counts = a[:, HID:HID + 1]
        out_ref[...] = a[:, :HID] / jnp.maximum(counts, 1.0)


def _pool_call(s2, y20, y21, dinv, b2, batch2d):
    return pl.pallas_call(
        _pool_body,
        grid=(GRID,),
        in_specs=[
            pl.BlockSpec((NCORE, NB, HALF), lambda i: (0, i, 0)),
            pl.BlockSpec((NB, HALF), lambda i: (i, 0)),
            pl.BlockSpec((NB, HALF), lambda i: (i, 0)),
            pl.BlockSpec((NB, 1), lambda i: (i, 0)),
            pl.BlockSpec((1, HID), lambda i: (0, 0)),
            pl.BlockSpec((1, 1, NB), lambda i: (i, 0, 0)),
        ],
        out_specs=pl.BlockSpec((G, HID), lambda i: (0, 0)),
        out_shape=jax.ShapeDtypeStruct((G, HID), _F32),
        scratch_shapes=[pltpu.VMEM((G, 2 * HID), _F32)],
    )(s2, y20, y21, dinv, b2, batch2d)


@functools.partial(jax.jit)
def kernel(x, edge_index, edge_attr, batch, W1, b1, W2, b2):
    del edge_attr
    src = edge_index[0].astype(jnp.int32)
    dst = edge_index[1].astype(jnp.int32)
    batch2d = batch.astype(jnp.int32).reshape(GRID, 1, NB)
    zrow1d = jnp.zeros((ROWS_PER_TILE,), _F32)
    zrows = jnp.zeros((STAGE_ROWS, HALF), _F32)
    b1r = b1.reshape(1, HID)
    b2r = b2.reshape(1, HID)

    pidx = jnp.concatenate(
        [src.reshape(E // SUP_EDGES, SUP_CHUNKS, MP_CHUNK),
         dst.reshape(E // SUP_EDGES, SUP_CHUNKS, MP_CHUNK)], axis=1)
    dstd = dst.reshape(E // DG_SUP_EDGES, DG_SUP, DG_CHUNK)
    degp = _deg_call(dstd, zrow1d)
    y0, y1, dinv = _prep1_call(degp.reshape(NCORE, N, 1), x, W1)
    s1 = _mp_call(y0, y1, pidx, zrows)
    y20, y21 = _prep2_call(s1, y0, y1, dinv, W2, b1r)
    s2 = _mp_call(y20, y21, pidx, zrows)
    return _pool_call(s2, y20, y21, dinv, b2r, batch2d)
